# Initial kernel scaffold; baseline (speedup 1.0000x reference)
import jax
import jax.numpy as jnp
from jax.experimental import pallas as pl


def kernel(x, edge_index, edge_weight, batch, W1, b1, W2, b2, W3, b3, gamma, beta, Wlin, blin):
    raise NotImplementedError("write your pallas kernel here")


# speedup vs baseline: 6.6516x; 6.6516x over previous
import functools

import jax
import jax.numpy as jnp
from jax import lax
from jax.experimental import pallas as pl
from jax.experimental.pallas import tpu as pltpu
from jax.experimental.pallas import tpu_sc as plsc

N = 10000
E = 320000
G = 16
D = 128
NC = 2
NS = 16
NW = NC * NS
EPW = E // NW
K = 80
NCHUNK = EPW // K
RPT = N // NS
TPW = 10240
ZR = 25
ZF = 2048


def _edge_pass_body(with_t, *refs):
    if with_t:
        (x_hbm, src_hbm, dst_hbm, w_hbm, batch_hbm,
         u_out, t_out,
         u_acc, srcv, dstv, wv, rows, zb2, sem,
         t_acc, bdstv, flatv, zb1) = refs
    else:
        (x_hbm, src_hbm, dst_hbm, w_hbm,
         u_out,
         u_acc, srcv, dstv, wv, rows, zb2, sem) = refs
        zb1 = None

    c = lax.axis_index("c")
    s = lax.axis_index("s")
    wid = c * NS + s

    def zrow(r, carry):
        for col in range(D // 16):
            zb2[r, pl.ds(col * 16, 16)] = jnp.zeros((16,), jnp.float32)
        return carry

    lax.fori_loop(0, ZR, zrow, 0)

    def zcopy_u(k, carry):
        pltpu.sync_copy(zb2, u_acc.at[pl.ds(s * RPT + k * ZR, ZR)])
        return carry

    lax.fori_loop(0, RPT // ZR, zcopy_u, 0)
    if with_t:
        def zflat(r, carry):
            zb1[pl.ds(r * 16, 16)] = jnp.zeros((16,), jnp.float32)
            return carry

        lax.fori_loop(0, ZF // 16, zflat, 0)

        def zcopy_t(k, carry):
            pltpu.sync_copy(zb1, t_acc.at[pl.ds(s * TPW + k * ZF, ZF)])
            return carry

        lax.fori_loop(0, TPW // ZF, zcopy_t, 0)
    plsc.subcore_barrier()

    def chunk(j, carry):
        base = wid * EPW + j * K
        pltpu.sync_copy(src_hbm.at[pl.ds(base, K)], srcv)
        pltpu.sync_copy(dst_hbm.at[pl.ds(base, K)], dstv)
        pltpu.sync_copy(w_hbm.at[pl.ds(base, K)], wv)
        pltpu.async_copy(x_hbm.at[srcv], rows, sem).wait()
        if with_t:
            pltpu.async_copy(batch_hbm.at[dstv], bdstv, sem).wait()
        for q in range(K // 16):
            wq = wv[pl.ds(q * 16, 16)]
            if with_t:
                bq = bdstv[pl.ds(q * 16, 16)]
                sq = srcv[pl.ds(q * 16, 16)]
                flatv[pl.ds(q * 16, 16)] = bq * N + sq
            for e in range(16):
                i = q * 16 + e
                splat = jnp.take_along_axis(
                    wq, jnp.full((16,), e, jnp.int32), axis=0)
                for col in range(D // 16):
                    sl = pl.ds(col * 16, 16)
                    rows[i, sl] = rows[i, sl] * splat
        pltpu.sync_copy(rows, u_acc.at[dstv], add=True)
        if with_t:
            pltpu.sync_copy(wv, t_acc.at[flatv], add=True)
        return carry

    lax.fori_loop(0, NCHUNK, chunk, 0)
    plsc.subcore_barrier()

    pltpu.sync_copy(u_acc.at[pl.ds(s * RPT, RPT)], u_out.at[c, s])
    if with_t:
        pltpu.sync_copy(t_acc.at[pl.ds(s * TPW, TPW)], t_out.at[c, s])


def _make_edge_pass(with_t):
    mesh = plsc.VectorSubcoreMesh(core_axis_name="c", subcore_axis_name="s")
    outs = [jax.ShapeDtypeStruct((NC, NS, RPT, D), jnp.float32)]
    scratch = [
        pltpu.VMEM_SHARED((N, D), jnp.float32),
        pltpu.VMEM((K,), jnp.int32),
        pltpu.VMEM((K,), jnp.int32),
        pltpu.VMEM((K,), jnp.float32),
        pltpu.VMEM((K, D), jnp.float32),
        pltpu.VMEM((ZR, D), jnp.float32),
        pltpu.SemaphoreType.DMA,
    ]
    if with_t:
        outs.append(jax.ShapeDtypeStruct((NC, NS, TPW), jnp.float32))
        scratch += [
            pltpu.VMEM_SHARED((NS * TPW,), jnp.float32),
            pltpu.VMEM((K,), jnp.int32),
            pltpu.VMEM((K,), jnp.int32),
            pltpu.VMEM((ZF,), jnp.float32),
        ]
    return pl.kernel(
        functools.partial(_edge_pass_body, with_t),
        out_type=outs,
        mesh=mesh,
        scratch_types=scratch,
    )


def _tc1_body(u_ref, W_ref, b_ref, h_ref, s_ref):
    sblk = u_ref[0] + u_ref[1]
    s_ref[...] = sblk
    h_ref[...] = jnp.maximum(
        jnp.dot(sblk, W_ref[...], preferred_element_type=jnp.float32,
                precision=lax.Precision.HIGHEST)
        + b_ref[...], 0.0)


def _tc2_body(u1_ref, s0_ref, h1_ref, t_ref, b_ref, W2_ref, b2_ref,
              P2o, P1o, Co, P2a, P1a, Ca):
    i = pl.program_id(0)

    @pl.when(i == 0)
    def _():
        P2a[...] = jnp.zeros_like(P2a)
        P1a[...] = jnp.zeros_like(P1a)
        Ca[...] = jnp.zeros_like(Ca)

    s1 = u1_ref[0] + u1_ref[1]
    hcat = jnp.concatenate([s1, s0_ref[...]], axis=1)
    h2 = jnp.maximum(
        jnp.dot(hcat, W2_ref[...], preferred_element_type=jnp.float32,
                precision=lax.Precision.HIGHEST)
        + b2_ref[...], 0.0)
    tb = t_ref[0, 0] + t_ref[1, 0]
    P2a[...] += jnp.dot(tb, h2, preferred_element_type=jnp.float32,
                precision=lax.Precision.HIGHEST)
    P1a[...] += jnp.dot(tb, h1_ref[...], preferred_element_type=jnp.float32,
                precision=lax.Precision.HIGHEST)
    onehot = (lax.broadcasted_iota(jnp.int32, (G, N // 10), 0)
              == b_ref[0]).astype(jnp.float32)
    Ca[...] += jnp.broadcast_to(
        jnp.sum(onehot, axis=1, keepdims=True), Ca.shape)
    P2o[...] = P2a[...]
    P1o[...] = P1a[...]
    Co[...] = Ca[...]


def _tc3_body(P2r, P1r, Cr, W3r, b3r, gr, br, Wlr, blr, outr):
    cnts = Cr[:, 0:1]
    pcat = jnp.concatenate([P2r[...], P1r[...]], axis=1)
    sums = (jnp.dot(pcat, W3r[...], preferred_element_type=jnp.float32,
                precision=lax.Precision.HIGHEST)
            + cnts * b3r[...])
    pooled = sums / jnp.maximum(cnts, 1.0)
    mean = jnp.mean(pooled, axis=0, keepdims=True)
    var = jnp.mean((pooled - mean) ** 2, axis=0, keepdims=True)
    xn = (pooled - mean) * lax.rsqrt(var + 1e-5) * gr[...] + br[...]
    outr[...] = (jnp.dot(xn, Wlr[...], preferred_element_type=jnp.float32,
                precision=lax.Precision.HIGHEST)
                 + blr[...])


_NB = 10
_BN = N // _NB


def kernel(x, edge_index, edge_weight, batch, W1, b1, W2, b2, W3, b3,
           gamma, beta, Wlin, blin):
    src = edge_index[0]
    dst = edge_index[1]

    u0p, tp = _make_edge_pass(True)(x, src, dst, edge_weight, batch)
    u0p = u0p.reshape(NC, N, D)
    tp = tp.reshape(NC, NS * TPW)[:, :G * N]

    h1, s0 = pl.pallas_call(
        _tc1_body,
        grid=(_NB,),
        in_specs=[
            pl.BlockSpec((NC, _BN, D), lambda i: (0, i, 0)),
            pl.BlockSpec((D, D), lambda i: (0, 0)),
            pl.BlockSpec((1, D), lambda i: (0, 0)),
        ],
        out_specs=[
            pl.BlockSpec((_BN, D), lambda i: (i, 0)),
            pl.BlockSpec((_BN, D), lambda i: (i, 0)),
        ],
        out_shape=[jax.ShapeDtypeStruct((N, D), jnp.float32)] * 2,
    )(u0p, W1, b1.reshape(1, D))

    u1p, = _make_edge_pass(False)(h1, src, dst, edge_weight)
    u1p = u1p.reshape(NC, N, D)

    t4 = tp.reshape(NC, G, _NB, _BN).transpose(0, 2, 1, 3)

    batch3 = batch.reshape(_NB, 1, _BN)
    C2 = 2 * D
    P2, P1, CNT = pl.pallas_call(
        _tc2_body,
        grid=(_NB,),
        in_specs=[
            pl.BlockSpec((NC, _BN, D), lambda i: (0, i, 0)),
            pl.BlockSpec((_BN, D), lambda i: (i, 0)),
            pl.BlockSpec((_BN, D), lambda i: (i, 0)),
            pl.BlockSpec((NC, 1, G, _BN), lambda i: (0, i, 0, 0)),
            pl.BlockSpec((1, 1, _BN), lambda i: (i, 0, 0)),
            pl.BlockSpec((C2, C2), lambda i: (0, 0)),
            pl.BlockSpec((1, C2), lambda i: (0, 0)),
        ],
        out_specs=[
            pl.BlockSpec((G, C2), lambda i: (0, 0)),
            pl.BlockSpec((G, D), lambda i: (0, 0)),
            pl.BlockSpec((G, D), lambda i: (0, 0)),
        ],
        out_shape=[
            jax.ShapeDtypeStruct((G, C2), jnp.float32),
            jax.ShapeDtypeStruct((G, D), jnp.float32),
            jax.ShapeDtypeStruct((G, D), jnp.float32),
        ],
        scratch_shapes=[
            pltpu.VMEM((G, C2), jnp.float32),
            pltpu.VMEM((G, D), jnp.float32),
            pltpu.VMEM((G, D), jnp.float32),
        ],
    )(u1p, s0, h1, t4, batch3, W2, b2.reshape(1, C2))

    C3 = 3 * D
    out = pl.pallas_call(
        _tc3_body,
        out_shape=jax.ShapeDtypeStruct((G, 1), jnp.float32),
    )(P2, P1, CNT, W3, b3.reshape(1, C3), gamma.reshape(1, C3),
      beta.reshape(1, C3), Wlin, blin.reshape(1, 1))
    return out


# speedup vs baseline: 15.2022x; 2.2855x over previous
import functools

import jax
import jax.numpy as jnp
from jax import lax
from jax.experimental import pallas as pl
from jax.experimental.pallas import tpu as pltpu
from jax.experimental.pallas import tpu_sc as plsc

N = 10000
E = 320000
G = 16
D = 128
NC = 2
NS = 16
NW = NC * NS
EPW = E // NW
K = 80
NCHUNK = EPW // K
RPT = N // NS
TPW = 10240
ZR = 25
ZF = 2048


def _edge_pass_body(with_t, *refs):
    if with_t:
        (x_hbm, pk3, w3, batch_hbm,
         u_out, t_out,
         u_acc, pkb, wb, dstsc, rows2, zb2,
         gsem0, gsem1, ssem0, ssem1, psem0, psem1, wsem0, wsem1,
         t_acc, wvbuf, bdst2, flat2, bsem0, bsem1, tsem0, tsem1,
         zb1) = refs
    else:
        (x_hbm, pk3, w3,
         u_out,
         u_acc, pkb, wb, dstsc, rows2, zb2,
         gsem0, gsem1, ssem0, ssem1, psem0, psem1, wsem0, wsem1) = refs
        zb1 = None

    c = lax.axis_index("c")
    s = lax.axis_index("s")
    wid = c * NS + s
    gsem = (gsem0, gsem1)
    ssem = (ssem0, ssem1)
    psem = (psem0, psem1)
    wsem = (wsem0, wsem1)
    if with_t:
        bsem = (bsem0, bsem1)
        tsem = (tsem0, tsem1)

    def zrow(r, carry):
        for col in range(D // 16):
            zb2[r, pl.ds(col * 16, 16)] = jnp.zeros((16,), jnp.float32)
        return carry

    lax.fori_loop(0, ZR, zrow, 0)

    def zcopy_u(k, carry):
        pltpu.sync_copy(zb2, u_acc.at[pl.ds(s * RPT + k * ZR, ZR)])
        return carry

    lax.fori_loop(0, RPT // ZR, zcopy_u, 0)
    if with_t:
        def zflat(r, carry):
            zb1[pl.ds(r * 16, 16)] = jnp.zeros((16,), jnp.float32)
            return carry

        lax.fori_loop(0, ZF // 16, zflat, 0)

        def zcopy_t(k, carry):
            pltpu.sync_copy(zb1, t_acc.at[pl.ds(s * TPW + k * ZF, ZF)])
            return carry

        lax.fori_loop(0, TPW // ZF, zcopy_t, 0)
    plsc.subcore_barrier()

    def issue_pack(cc, p):
        pltpu.async_copy(pk3.at[wid, cc], pkb.at[p], psem[p])
        pltpu.async_copy(w3.at[wid, cc], wb.at[p], wsem[p])

    def wait_pack(cc, p):
        pltpu.make_async_copy(pk3.at[wid, cc], pkb.at[p], psem[p]).wait()
        pltpu.make_async_copy(w3.at[wid, cc], wb.at[p], wsem[p]).wait()

    def issue_gather(p):
        pltpu.async_copy(x_hbm.at[pkb.at[p, 0]], rows2.at[p], gsem[p])
        if with_t:
            pltpu.async_copy(batch_hbm.at[pkb.at[p, 1]], bdst2.at[p],
                             bsem[p])

    def wait_gather(p):
        pltpu.make_async_copy(
            x_hbm.at[pkb.at[p, 0]], rows2.at[p], gsem[p]).wait()
        if with_t:
            pltpu.make_async_copy(
                batch_hbm.at[pkb.at[p, 1]], bdst2.at[p], bsem[p]).wait()

    def wait_scatter(p):
        pltpu.make_async_copy(
            rows2.at[p], u_acc.at[dstsc.at[p]], ssem[p]).wait()

    def wait_tscatter(p):
        pltpu.make_async_copy(
            wvbuf.at[p], t_acc.at[flat2.at[p]], tsem[p]).wait()

    def half(cc, p):
        q = 1 - p
        wait_gather(p)
        @pl.when(cc >= 1)
        def _():
            wait_scatter(q)
            if with_t:
                wait_tscatter(q)

        @pl.when(cc + 1 < NCHUNK)
        def _():
            wait_pack(cc + 1, q)
            issue_gather(q)

        for q5 in range(K // 16):
            sl16 = pl.ds(q5 * 16, 16)
            wq = wb[p, sl16]
            dstsc[p, sl16] = pkb[p, 1, sl16]
            if with_t:
                wvbuf[p, sl16] = wq
                flat2[p, sl16] = bdst2[p, sl16] * N + pkb[p, 0, sl16]
            for e in range(16):
                i = q5 * 16 + e
                splat = jnp.take_along_axis(
                    wq, jnp.full((16,), e, jnp.int32), axis=0)
                for col in range(D // 16):
                    sl = pl.ds(col * 16, 16)
                    rows2[p, i, sl] = rows2[p, i, sl] * splat

        @pl.when(cc + 2 < NCHUNK)
        def _():
            issue_pack(cc + 2, p)

        pltpu.async_copy(rows2.at[p], u_acc.at[dstsc.at[p]], ssem[p],
                         add=True)
        if with_t:
            pltpu.async_copy(wvbuf.at[p], t_acc.at[flat2.at[p]], tsem[p],
                             add=True)

    pltpu.sync_copy(pk3.at[wid, 0], pkb.at[0])
    pltpu.sync_copy(w3.at[wid, 0], wb.at[0])
    issue_pack(1, 1)
    issue_gather(0)

    def pair(jj, carry):
        half(2 * jj, 0)

        @pl.when(2 * jj + 1 < NCHUNK)
        def _():
            half(2 * jj + 1, 1)
        return carry

    lax.fori_loop(0, (NCHUNK + 1) // 2, pair, 0)
    wait_scatter((NCHUNK - 1) % 2)
    if with_t:
        wait_tscatter((NCHUNK - 1) % 2)
    plsc.subcore_barrier()

    pltpu.sync_copy(u_acc.at[pl.ds(s * RPT, RPT)], u_out.at[c, s])
    if with_t:
        pltpu.sync_copy(t_acc.at[pl.ds(s * TPW, TPW)], t_out.at[c, s])


def _make_edge_pass(with_t):
    mesh = plsc.VectorSubcoreMesh(core_axis_name="c", subcore_axis_name="s")
    outs = [jax.ShapeDtypeStruct((NC, NS, RPT, D), jnp.float32)]
    scratch = [
        pltpu.VMEM_SHARED((N, D), jnp.float32),
        pltpu.VMEM((2, 2, K), jnp.int32),
        pltpu.VMEM((2, K), jnp.float32),
        pltpu.VMEM((2, K), jnp.int32),
        pltpu.VMEM((2, K, D), jnp.float32),
        pltpu.VMEM((ZR, D), jnp.float32),
        pltpu.SemaphoreType.DMA,
        pltpu.SemaphoreType.DMA,
        pltpu.SemaphoreType.DMA,
        pltpu.SemaphoreType.DMA,
        pltpu.SemaphoreType.DMA,
        pltpu.SemaphoreType.DMA,
        pltpu.SemaphoreType.DMA,
        pltpu.SemaphoreType.DMA,
    ]
    if with_t:
        outs.append(jax.ShapeDtypeStruct((NC, NS, TPW), jnp.float32))
        scratch += [
            pltpu.VMEM_SHARED((NS * TPW,), jnp.float32),
            pltpu.VMEM((2, K), jnp.float32),
            pltpu.VMEM((2, K), jnp.int32),
            pltpu.VMEM((2, K), jnp.int32),
            pltpu.SemaphoreType.DMA,
            pltpu.SemaphoreType.DMA,
            pltpu.SemaphoreType.DMA,
            pltpu.SemaphoreType.DMA,
            pltpu.VMEM((ZF,), jnp.float32),
        ]
    return pl.kernel(
        functools.partial(_edge_pass_body, with_t),
        out_type=outs,
        mesh=mesh,
        scratch_types=scratch,
    )


def _tc1_body(u_ref, W_ref, b_ref, h_ref, s_ref):
    sblk = u_ref[0] + u_ref[1]
    s_ref[...] = sblk
    h_ref[...] = jnp.maximum(
        jnp.dot(sblk, W_ref[...], preferred_element_type=jnp.float32,
                precision=lax.Precision.HIGHEST)
        + b_ref[...], 0.0)


def _tc2_body(u1_ref, s0_ref, h1_ref, t_ref, b_ref, W2_ref, b2_ref,
              P2o, P1o, Co, P2a, P1a, Ca):
    i = pl.program_id(0)

    @pl.when(i == 0)
    def _():
        P2a[...] = jnp.zeros_like(P2a)
        P1a[...] = jnp.zeros_like(P1a)
        Ca[...] = jnp.zeros_like(Ca)

    s1 = u1_ref[0] + u1_ref[1]
    hcat = jnp.concatenate([s1, s0_ref[...]], axis=1)
    h2 = jnp.maximum(
        jnp.dot(hcat, W2_ref[...], preferred_element_type=jnp.float32,
                precision=lax.Precision.HIGHEST)
        + b2_ref[...], 0.0)
    tb = t_ref[0, 0] + t_ref[1, 0]
    P2a[...] += jnp.dot(tb, h2, preferred_element_type=jnp.float32,
                precision=lax.Precision.HIGHEST)
    P1a[...] += jnp.dot(tb, h1_ref[...], preferred_element_type=jnp.float32,
                precision=lax.Precision.HIGHEST)
    onehot = (lax.broadcasted_iota(jnp.int32, (G, N // 10), 0)
              == b_ref[0]).astype(jnp.float32)
    Ca[...] += jnp.broadcast_to(
        jnp.sum(onehot, axis=1, keepdims=True), Ca.shape)
    P2o[...] = P2a[...]
    P1o[...] = P1a[...]
    Co[...] = Ca[...]


def _tc3_body(P2r, P1r, Cr, W3r, b3r, gr, br, Wlr, blr, outr):
    cnts = Cr[:, 0:1]
    pcat = jnp.concatenate([P2r[...], P1r[...]], axis=1)
    sums = (jnp.dot(pcat, W3r[...], preferred_element_type=jnp.float32,
                precision=lax.Precision.HIGHEST)
            + cnts * b3r[...])
    pooled = sums / jnp.maximum(cnts, 1.0)
    mean = jnp.mean(pooled, axis=0, keepdims=True)
    var = jnp.mean((pooled - mean) ** 2, axis=0, keepdims=True)
    xn = (pooled - mean) * lax.rsqrt(var + 1e-5) * gr[...] + br[...]
    outr[...] = (jnp.dot(xn, Wlr[...], preferred_element_type=jnp.float32,
                precision=lax.Precision.HIGHEST)
                 + blr[...])


_NB = 10
_BN = N // _NB


def kernel(x, edge_index, edge_weight, batch, W1, b1, W2, b2, W3, b3,
           gamma, beta, Wlin, blin):
    pk3 = jnp.stack(
        [edge_index[0].reshape(NW, NCHUNK, K),
         edge_index[1].reshape(NW, NCHUNK, K)],
        axis=2)
    w3 = edge_weight.reshape(NW, NCHUNK, K)

    u0p, tp = _make_edge_pass(True)(x, pk3, w3, batch)
    u0p = u0p.reshape(NC, N, D)
    tp = tp.reshape(NC, NS * TPW)[:, :G * N]

    h1, s0 = pl.pallas_call(
        _tc1_body,
        grid=(_NB,),
        in_specs=[
            pl.BlockSpec((NC, _BN, D), lambda i: (0, i, 0)),
            pl.BlockSpec((D, D), lambda i: (0, 0)),
            pl.BlockSpec((1, D), lambda i: (0, 0)),
        ],
        out_specs=[
            pl.BlockSpec((_BN, D), lambda i: (i, 0)),
            pl.BlockSpec((_BN, D), lambda i: (i, 0)),
        ],
        out_shape=[jax.ShapeDtypeStruct((N, D), jnp.float32)] * 2,
    )(u0p, W1, b1.reshape(1, D))

    u1p, = _make_edge_pass(False)(h1, pk3, w3)
    u1p = u1p.reshape(NC, N, D)

    t4 = tp.reshape(NC, G, _NB, _BN).transpose(0, 2, 1, 3)

    batch3 = batch.reshape(_NB, 1, _BN)
    C2 = 2 * D
    P2, P1, CNT = pl.pallas_call(
        _tc2_body,
        grid=(_NB,),
        in_specs=[
            pl.BlockSpec((NC, _BN, D), lambda i: (0, i, 0)),
            pl.BlockSpec((_BN, D), lambda i: (i, 0)),
            pl.BlockSpec((_BN, D), lambda i: (i, 0)),
            pl.BlockSpec((NC, 1, G, _BN), lambda i: (0, i, 0, 0)),
            pl.BlockSpec((1, 1, _BN), lambda i: (i, 0, 0)),
            pl.BlockSpec((C2, C2), lambda i: (0, 0)),
            pl.BlockSpec((1, C2), lambda i: (0, 0)),
        ],
        out_specs=[
            pl.BlockSpec((G, C2), lambda i: (0, 0)),
            pl.BlockSpec((G, D), lambda i: (0, 0)),
            pl.BlockSpec((G, D), lambda i: (0, 0)),
        ],
        out_shape=[
            jax.ShapeDtypeStruct((G, C2), jnp.float32),
            jax.ShapeDtypeStruct((G, D), jnp.float32),
            jax.ShapeDtypeStruct((G, D), jnp.float32),
        ],
        scratch_shapes=[
            pltpu.VMEM((G, C2), jnp.float32),
            pltpu.VMEM((G, D), jnp.float32),
            pltpu.VMEM((G, D), jnp.float32),
        ],
    )(u1p, s0, h1, t4, batch3, W2, b2.reshape(1, C2))

    C3 = 3 * D
    out = pl.pallas_call(
        _tc3_body,
        out_shape=jax.ShapeDtypeStruct((G, 1), jnp.float32),
    )(P2, P1, CNT, W3, b3.reshape(1, C3), gamma.reshape(1, C3),
      beta.reshape(1, C3), Wlin, blin.reshape(1, 1))
    return out


# speedup vs baseline: 15.2822x; 1.0053x over previous
import functools

import jax
import jax.numpy as jnp
from jax import lax
from jax.experimental import pallas as pl
from jax.experimental.pallas import tpu as pltpu
from jax.experimental.pallas import tpu_sc as plsc

N = 10000
E = 320000
G = 16
D = 128
NC = 2
NS = 16
NW = NC * NS
EPW = E // NW
K = 80
NCHUNK = EPW // K
RPT = N // NS
TPW = 10240
ZR = 25
ZF = 2048


def _edge_pass_body(with_t, *refs):
    if with_t:
        (x_hbm, pk3, w3, batch_hbm,
         u_out, t_out,
         u_acc, pkb, wb, dstsc, rows2, zb2,
         gsem0, gsem1, ssem0, ssem1, psem0, psem1, wsem0, wsem1,
         t_acc, wvbuf, bdst2, flat2, bsem0, bsem1, tsem0, tsem1,
         zb1) = refs
    else:
        (x_hbm, pk3, w3,
         u_out,
         u_acc, pkb, wb, dstsc, rows2, zb2,
         gsem0, gsem1, ssem0, ssem1, psem0, psem1, wsem0, wsem1) = refs
        zb1 = None

    c = lax.axis_index("c")
    s = lax.axis_index("s")
    wid = c * NS + s
    gsem = (gsem0, gsem1)
    ssem = (ssem0, ssem1)
    psem = (psem0, psem1)
    wsem = (wsem0, wsem1)
    if with_t:
        bsem = (bsem0, bsem1)
        tsem = (tsem0, tsem1)

    def zrow(r, carry):
        for col in range(D // 16):
            zb2[r, pl.ds(col * 16, 16)] = jnp.zeros((16,), jnp.float32)
        return carry

    lax.fori_loop(0, ZR, zrow, 0)

    def zcopy_u(k, carry):
        pltpu.sync_copy(zb2, u_acc.at[pl.ds(s * RPT + k * ZR, ZR)])
        return carry

    lax.fori_loop(0, RPT // ZR, zcopy_u, 0)
    if with_t:
        def zflat(r, carry):
            zb1[pl.ds(r * 16, 16)] = jnp.zeros((16,), jnp.float32)
            return carry

        lax.fori_loop(0, ZF // 16, zflat, 0)

        def zcopy_t(k, carry):
            pltpu.sync_copy(zb1, t_acc.at[pl.ds(s * TPW + k * ZF, ZF)])
            return carry

        lax.fori_loop(0, TPW // ZF, zcopy_t, 0)
    plsc.subcore_barrier()

    def issue_pack(cc, p):
        pltpu.async_copy(pk3.at[wid, cc], pkb.at[p], psem[p])
        pltpu.async_copy(w3.at[wid, cc], wb.at[p], wsem[p])

    def wait_pack(cc, p):
        pltpu.make_async_copy(pk3.at[wid, cc], pkb.at[p], psem[p]).wait()
        pltpu.make_async_copy(w3.at[wid, cc], wb.at[p], wsem[p]).wait()

    def issue_gather(p):
        pltpu.async_copy(x_hbm.at[pkb.at[p, 0]], rows2.at[p], gsem[p])
        if with_t:
            pltpu.async_copy(batch_hbm.at[pkb.at[p, 1]], bdst2.at[p],
                             bsem[p])

    def wait_gather(p):
        pltpu.make_async_copy(
            x_hbm.at[pkb.at[p, 0]], rows2.at[p], gsem[p]).wait()
        if with_t:
            pltpu.make_async_copy(
                batch_hbm.at[pkb.at[p, 1]], bdst2.at[p], bsem[p]).wait()

    def wait_scatter(p):
        pltpu.make_async_copy(
            rows2.at[p], u_acc.at[dstsc.at[p]], ssem[p]).wait()

    def wait_tscatter(p):
        pltpu.make_async_copy(
            wvbuf.at[p], t_acc.at[flat2.at[p]], tsem[p]).wait()

    def half(cc, p):
        q = 1 - p
        wait_gather(p)
        @pl.when(cc >= 1)
        def _():
            wait_scatter(q)
            if with_t:
                wait_tscatter(q)

        @pl.when(cc + 1 < NCHUNK)
        def _():
            wait_pack(cc + 1, q)
            issue_gather(q)

        @plsc.parallel_loop(0, K // 16, 1, unroll=K // 16)
        def _scale(q5):
            sl16 = pl.ds(q5 * 16, 16)
            wq = wb[p, sl16]
            dstsc[p, sl16] = pkb[p, 1, sl16]
            if with_t:
                wvbuf[p, sl16] = wq
                flat2[p, sl16] = bdst2[p, sl16] * N + pkb[p, 0, sl16]
            for e in range(16):
                i = q5 * 16 + e
                splat = jnp.take_along_axis(
                    wq, jnp.full((16,), e, jnp.int32), axis=0)
                for col in range(D // 16):
                    sl = pl.ds(col * 16, 16)
                    rows2[p, i, sl] = rows2[p, i, sl] * splat

        @pl.when(cc + 2 < NCHUNK)
        def _():
            issue_pack(cc + 2, p)

        pltpu.async_copy(rows2.at[p], u_acc.at[dstsc.at[p]], ssem[p],
                         add=True)
        if with_t:
            pltpu.async_copy(wvbuf.at[p], t_acc.at[flat2.at[p]], tsem[p],
                             add=True)

    pltpu.sync_copy(pk3.at[wid, 0], pkb.at[0])
    pltpu.sync_copy(w3.at[wid, 0], wb.at[0])
    issue_pack(1, 1)
    issue_gather(0)

    def pair(jj, carry):
        half(2 * jj, 0)

        @pl.when(2 * jj + 1 < NCHUNK)
        def _():
            half(2 * jj + 1, 1)
        return carry

    lax.fori_loop(0, (NCHUNK + 1) // 2, pair, 0)
    wait_scatter((NCHUNK - 1) % 2)
    if with_t:
        wait_tscatter((NCHUNK - 1) % 2)
    plsc.subcore_barrier()

    pltpu.sync_copy(u_acc.at[pl.ds(s * RPT, RPT)], u_out.at[c, s])
    if with_t:
        pltpu.sync_copy(t_acc.at[pl.ds(s * TPW, TPW)], t_out.at[c, s])


def _make_edge_pass(with_t):
    mesh = plsc.VectorSubcoreMesh(core_axis_name="c", subcore_axis_name="s")
    outs = [jax.ShapeDtypeStruct((NC, NS, RPT, D), jnp.float32)]
    scratch = [
        pltpu.VMEM_SHARED((N, D), jnp.float32),
        pltpu.VMEM((2, 2, K), jnp.int32),
        pltpu.VMEM((2, K), jnp.float32),
        pltpu.VMEM((2, K), jnp.int32),
        pltpu.VMEM((2, K, D), jnp.float32),
        pltpu.VMEM((ZR, D), jnp.float32),
        pltpu.SemaphoreType.DMA,
        pltpu.SemaphoreType.DMA,
        pltpu.SemaphoreType.DMA,
        pltpu.SemaphoreType.DMA,
        pltpu.SemaphoreType.DMA,
        pltpu.SemaphoreType.DMA,
        pltpu.SemaphoreType.DMA,
        pltpu.SemaphoreType.DMA,
    ]
    if with_t:
        outs.append(jax.ShapeDtypeStruct((NC, NS, TPW), jnp.float32))
        scratch += [
            pltpu.VMEM_SHARED((NS * TPW,), jnp.float32),
            pltpu.VMEM((2, K), jnp.float32),
            pltpu.VMEM((2, K), jnp.int32),
            pltpu.VMEM((2, K), jnp.int32),
            pltpu.SemaphoreType.DMA,
            pltpu.SemaphoreType.DMA,
            pltpu.SemaphoreType.DMA,
            pltpu.SemaphoreType.DMA,
            pltpu.VMEM((ZF,), jnp.float32),
        ]
    return pl.kernel(
        functools.partial(_edge_pass_body, with_t),
        out_type=outs,
        mesh=mesh,
        scratch_types=scratch,
    )


def _tc1_body(u_ref, W_ref, b_ref, h_ref, s_ref):
    sblk = u_ref[0] + u_ref[1]
    s_ref[...] = sblk
    h_ref[...] = jnp.maximum(
        jnp.dot(sblk, W_ref[...], preferred_element_type=jnp.float32,
                precision=lax.Precision.HIGHEST)
        + b_ref[...], 0.0)


def _tc2_body(u1_ref, s0_ref, h1_ref, t_ref, b_ref, W2_ref, b2_ref,
              W3r, b3r, gr, br, Wlr, blr,
              outr, P2a, P1a, Ca):
    i = pl.program_id(0)

    @pl.when(i == 0)
    def _():
        P2a[...] = jnp.zeros_like(P2a)
        P1a[...] = jnp.zeros_like(P1a)
        Ca[...] = jnp.zeros_like(Ca)

    s1 = u1_ref[0] + u1_ref[1]
    hcat = jnp.concatenate([s1, s0_ref[...]], axis=1)
    h2 = jnp.maximum(
        jnp.dot(hcat, W2_ref[...], preferred_element_type=jnp.float32,
                precision=lax.Precision.HIGHEST)
        + b2_ref[...], 0.0)
    tb = t_ref[0, 0] + t_ref[1, 0]
    P2a[...] += jnp.dot(tb, h2, preferred_element_type=jnp.float32,
                precision=lax.Precision.HIGHEST)
    P1a[...] += jnp.dot(tb, h1_ref[...], preferred_element_type=jnp.float32,
                precision=lax.Precision.HIGHEST)
    onehot = (lax.broadcasted_iota(jnp.int32, (G, _BN), 0)
              == b_ref[0]).astype(jnp.float32)
    Ca[...] += jnp.broadcast_to(
        jnp.sum(onehot, axis=1, keepdims=True), Ca.shape)

    @pl.when(i == _NB - 1)
    def _():
        cnts = Ca[:, 0:1]
        pcat = jnp.concatenate([P2a[...], P1a[...]], axis=1)
        sums = (jnp.dot(pcat, W3r[...], preferred_element_type=jnp.float32,
                        precision=lax.Precision.HIGHEST)
                + cnts * b3r[...])
        pooled = sums / jnp.maximum(cnts, 1.0)
        mean = jnp.mean(pooled, axis=0, keepdims=True)
        var = jnp.mean((pooled - mean) ** 2, axis=0, keepdims=True)
        xn = (pooled - mean) * lax.rsqrt(var + 1e-5) * gr[...] + br[...]
        outr[...] = (jnp.dot(xn, Wlr[...],
                             preferred_element_type=jnp.float32,
                             precision=lax.Precision.HIGHEST)
                     + blr[...])


_NB = 10
_BN = N // _NB


def kernel(x, edge_index, edge_weight, batch, W1, b1, W2, b2, W3, b3,
           gamma, beta, Wlin, blin):
    pk3 = jnp.stack(
        [edge_index[0].reshape(NW, NCHUNK, K),
         edge_index[1].reshape(NW, NCHUNK, K)],
        axis=2)
    w3 = edge_weight.reshape(NW, NCHUNK, K)

    u0p, tp = _make_edge_pass(True)(x, pk3, w3, batch)
    u0p = u0p.reshape(NC, N, D)
    tp = tp.reshape(NC, NS * TPW)[:, :G * N]

    h1, s0 = pl.pallas_call(
        _tc1_body,
        grid=(_NB,),
        in_specs=[
            pl.BlockSpec((NC, _BN, D), lambda i: (0, i, 0)),
            pl.BlockSpec((D, D), lambda i: (0, 0)),
            pl.BlockSpec((1, D), lambda i: (0, 0)),
        ],
        out_specs=[
            pl.BlockSpec((_BN, D), lambda i: (i, 0)),
            pl.BlockSpec((_BN, D), lambda i: (i, 0)),
        ],
        out_shape=[jax.ShapeDtypeStruct((N, D), jnp.float32)] * 2,
    )(u0p, W1, b1.reshape(1, D))

    u1p, = _make_edge_pass(False)(h1, pk3, w3)
    u1p = u1p.reshape(NC, N, D)

    t4 = tp.reshape(NC, G, _NB, _BN).transpose(0, 2, 1, 3)

    batch3 = batch.reshape(_NB, 1, _BN)
    C2 = 2 * D
    C3 = 3 * D
    out = pl.pallas_call(
        _tc2_body,
        grid=(_NB,),
        in_specs=[
            pl.BlockSpec((NC, _BN, D), lambda i: (0, i, 0)),
            pl.BlockSpec((_BN, D), lambda i: (i, 0)),
            pl.BlockSpec((_BN, D), lambda i: (i, 0)),
            pl.BlockSpec((NC, 1, G, _BN), lambda i: (0, i, 0, 0)),
            pl.BlockSpec((1, 1, _BN), lambda i: (i, 0, 0)),
            pl.BlockSpec((C2, C2), lambda i: (0, 0)),
            pl.BlockSpec((1, C2), lambda i: (0, 0)),
            pl.BlockSpec((C3, C3), lambda i: (0, 0)),
            pl.BlockSpec((1, C3), lambda i: (0, 0)),
            pl.BlockSpec((1, C3), lambda i: (0, 0)),
            pl.BlockSpec((1, C3), lambda i: (0, 0)),
            pl.BlockSpec((C3, 1), lambda i: (0, 0)),
            pl.BlockSpec((1, 1), lambda i: (0, 0)),
        ],
        out_specs=pl.BlockSpec((G, 1), lambda i: (0, 0)),
        out_shape=jax.ShapeDtypeStruct((G, 1), jnp.float32),
        scratch_shapes=[
            pltpu.VMEM((G, C2), jnp.float32),
            pltpu.VMEM((G, D), jnp.float32),
            pltpu.VMEM((G, D), jnp.float32),
        ],
    )(u1p, s0, h1, t4, batch3, W2, b2.reshape(1, C2),
      W3, b3.reshape(1, C3), gamma.reshape(1, C3), beta.reshape(1, C3),
      Wlin, blin.reshape(1, 1))
    return out


# speedup vs baseline: 16.0952x; 1.0532x over previous
import functools

import jax
import jax.numpy as jnp
from jax import lax
from jax.experimental import pallas as pl
from jax.experimental.pallas import tpu as pltpu
from jax.experimental.pallas import tpu_sc as plsc

N = 10000
E = 320000
G = 16
D = 128
NC = 2
NS = 16
NW = NC * NS
EPW = E // NW
K = 80
NCHUNK = EPW // K
RPT = N // NS
TPW = 10240
ZR = 25
ZF = 2048


def _edge_pass_body(with_t, *refs):
    if with_t:
        (x_hbm, pk3, w3, batch_hbm,
         u_out, t_out,
         u_acc, pkb, wb, dstsc, rows2, zb2,
         gsem0, gsem1, ssem0, ssem1, psem0, psem1, wsem0, wsem1,
         t_acc, wvbuf, bdst2, flat2, bsem0, bsem1, tsem0, tsem1,
         zb1) = refs
    else:
        (x_hbm, pk3, w3,
         u_out,
         u_acc, pkb, wb, dstsc, rows2, zb2,
         gsem0, gsem1, ssem0, ssem1, psem0, psem1, wsem0, wsem1) = refs
        zb1 = None

    c = lax.axis_index("c")
    s = lax.axis_index("s")
    wid = c * NS + s
    gsem = (gsem0, gsem1)
    ssem = (ssem0, ssem1)
    psem = (psem0, psem1)
    wsem = (wsem0, wsem1)
    if with_t:
        bsem = (bsem0, bsem1)
        tsem = (tsem0, tsem1)

    def zrow(r, carry):
        for col in range(D // 16):
            zb2[r, pl.ds(col * 16, 16)] = jnp.zeros((16,), jnp.float32)
        return carry

    lax.fori_loop(0, ZR, zrow, 0)

    def zcopy_u(k, carry):
        pltpu.sync_copy(zb2, u_acc.at[pl.ds(s * RPT + k * ZR, ZR)])
        return carry

    lax.fori_loop(0, RPT // ZR, zcopy_u, 0)
    if with_t:
        def zflat(r, carry):
            zb1[pl.ds(r * 16, 16)] = jnp.zeros((16,), jnp.float32)
            return carry

        lax.fori_loop(0, ZF // 16, zflat, 0)

        def zcopy_t(k, carry):
            pltpu.sync_copy(zb1, t_acc.at[pl.ds(s * TPW + k * ZF, ZF)])
            return carry

        lax.fori_loop(0, TPW // ZF, zcopy_t, 0)
    plsc.subcore_barrier()

    def issue_pack(cc, p):
        pltpu.async_copy(pk3.at[0, wid, cc], pkb.at[p, 0], psem[p])
        pltpu.async_copy(pk3.at[1, wid, cc], pkb.at[p, 1], psem[p])
        pltpu.async_copy(w3.at[wid, cc], wb.at[p], wsem[p])

    def wait_pack(cc, p):
        pltpu.make_async_copy(pk3.at[0, wid, cc], pkb.at[p, 0], psem[p]).wait()
        pltpu.make_async_copy(pk3.at[1, wid, cc], pkb.at[p, 1], psem[p]).wait()
        pltpu.make_async_copy(w3.at[wid, cc], wb.at[p], wsem[p]).wait()

    def issue_gather(p):
        pltpu.async_copy(x_hbm.at[pkb.at[p, 0]], rows2.at[p], gsem[p])
        if with_t:
            pltpu.async_copy(batch_hbm.at[pkb.at[p, 1]], bdst2.at[p],
                             bsem[p])

    def wait_gather(p):
        pltpu.make_async_copy(
            x_hbm.at[pkb.at[p, 0]], rows2.at[p], gsem[p]).wait()
        if with_t:
            pltpu.make_async_copy(
                batch_hbm.at[pkb.at[p, 1]], bdst2.at[p], bsem[p]).wait()

    def wait_scatter(p):
        pltpu.make_async_copy(
            rows2.at[p], u_acc.at[dstsc.at[p]], ssem[p]).wait()

    def wait_tscatter(p):
        pltpu.make_async_copy(
            wvbuf.at[p], t_acc.at[flat2.at[p]], tsem[p]).wait()

    def half(cc, p):
        q = 1 - p
        wait_gather(p)
        @pl.when(cc >= 1)
        def _():
            wait_scatter(q)
            if with_t:
                wait_tscatter(q)

        @pl.when(cc + 1 < NCHUNK)
        def _():
            wait_pack(cc + 1, q)
            issue_gather(q)

        @plsc.parallel_loop(0, K // 16, 1, unroll=K // 16)
        def _scale(q5):
            sl16 = pl.ds(q5 * 16, 16)
            wq = wb[p, sl16]
            dstsc[p, sl16] = pkb[p, 1, sl16]
            if with_t:
                wvbuf[p, sl16] = wq
                flat2[p, sl16] = bdst2[p, sl16] * N + pkb[p, 0, sl16]
            for e in range(16):
                i = q5 * 16 + e
                splat = jnp.take_along_axis(
                    wq, jnp.full((16,), e, jnp.int32), axis=0)
                for col in range(D // 16):
                    sl = pl.ds(col * 16, 16)
                    rows2[p, i, sl] = rows2[p, i, sl] * splat

        @pl.when(cc + 2 < NCHUNK)
        def _():
            issue_pack(cc + 2, p)

        pltpu.async_copy(rows2.at[p], u_acc.at[dstsc.at[p]], ssem[p],
                         add=True)
        if with_t:
            pltpu.async_copy(wvbuf.at[p], t_acc.at[flat2.at[p]], tsem[p],
                             add=True)

    pltpu.sync_copy(pk3.at[0, wid, 0], pkb.at[0, 0])
    pltpu.sync_copy(pk3.at[1, wid, 0], pkb.at[0, 1])
    pltpu.sync_copy(w3.at[wid, 0], wb.at[0])
    issue_pack(1, 1)
    issue_gather(0)

    def pair(jj, carry):
        half(2 * jj, 0)

        @pl.when(2 * jj + 1 < NCHUNK)
        def _():
            half(2 * jj + 1, 1)
        return carry

    lax.fori_loop(0, (NCHUNK + 1) // 2, pair, 0)
    wait_scatter((NCHUNK - 1) % 2)
    if with_t:
        wait_tscatter((NCHUNK - 1) % 2)
    plsc.subcore_barrier()

    pltpu.sync_copy(u_acc.at[pl.ds(s * RPT, RPT)], u_out.at[c, s])
    if with_t:
        pltpu.sync_copy(t_acc.at[pl.ds(s * TPW, TPW)], t_out.at[c, s])


def _make_edge_pass(with_t):
    mesh = plsc.VectorSubcoreMesh(core_axis_name="c", subcore_axis_name="s")
    outs = [jax.ShapeDtypeStruct((NC, NS, RPT, D), jnp.float32)]
    scratch = [
        pltpu.VMEM_SHARED((N, D), jnp.float32),
        pltpu.VMEM((2, 2, K), jnp.int32),
        pltpu.VMEM((2, K), jnp.float32),
        pltpu.VMEM((2, K), jnp.int32),
        pltpu.VMEM((2, K, D), jnp.float32),
        pltpu.VMEM((ZR, D), jnp.float32),
        pltpu.SemaphoreType.DMA,
        pltpu.SemaphoreType.DMA,
        pltpu.SemaphoreType.DMA,
        pltpu.SemaphoreType.DMA,
        pltpu.SemaphoreType.DMA,
        pltpu.SemaphoreType.DMA,
        pltpu.SemaphoreType.DMA,
        pltpu.SemaphoreType.DMA,
    ]
    if with_t:
        outs.append(jax.ShapeDtypeStruct((NC, NS, TPW), jnp.float32))
        scratch += [
            pltpu.VMEM_SHARED((NS * TPW,), jnp.float32),
            pltpu.VMEM((2, K), jnp.float32),
            pltpu.VMEM((2, K), jnp.int32),
            pltpu.VMEM((2, K), jnp.int32),
            pltpu.SemaphoreType.DMA,
            pltpu.SemaphoreType.DMA,
            pltpu.SemaphoreType.DMA,
            pltpu.SemaphoreType.DMA,
            pltpu.VMEM((ZF,), jnp.float32),
        ]
    return pl.kernel(
        functools.partial(_edge_pass_body, with_t),
        out_type=outs,
        mesh=mesh,
        scratch_types=scratch,
    )


def _tc1_body(u_ref, W_ref, b_ref, h_ref, s_ref):
    sblk = u_ref[0] + u_ref[1]
    s_ref[...] = sblk
    h_ref[...] = jnp.maximum(
        jnp.dot(sblk, W_ref[...], preferred_element_type=jnp.float32,
                precision=lax.Precision.HIGHEST)
        + b_ref[...], 0.0)


def _tc2_body(u1_ref, s0_ref, h1_ref, t_ref, b_ref, W2_ref, b2_ref,
              W3r, b3r, gr, br, Wlr, blr,
              outr, P2a, P1a, Ca):
    i = pl.program_id(0)

    @pl.when(i == 0)
    def _():
        P2a[...] = jnp.zeros_like(P2a)
        P1a[...] = jnp.zeros_like(P1a)
        Ca[...] = jnp.zeros_like(Ca)

    s1 = u1_ref[0] + u1_ref[1]
    hcat = jnp.concatenate([s1, s0_ref[...]], axis=1)
    h2 = jnp.maximum(
        jnp.dot(hcat, W2_ref[...], preferred_element_type=jnp.float32,
                precision=lax.Precision.HIGHEST)
        + b2_ref[...], 0.0)
    tb = t_ref[0, 0] + t_ref[1, 0]
    P2a[...] += jnp.dot(tb, h2, preferred_element_type=jnp.float32,
                precision=lax.Precision.HIGHEST)
    P1a[...] += jnp.dot(tb, h1_ref[...], preferred_element_type=jnp.float32,
                precision=lax.Precision.HIGHEST)
    onehot = (lax.broadcasted_iota(jnp.int32, (G, _BN), 0)
              == b_ref[0]).astype(jnp.float32)
    Ca[...] += jnp.broadcast_to(
        jnp.sum(onehot, axis=1, keepdims=True), Ca.shape)

    @pl.when(i == _NB - 1)
    def _():
        cnts = Ca[:, 0:1]
        pcat = jnp.concatenate([P2a[...], P1a[...]], axis=1)
        sums = (jnp.dot(pcat, W3r[...], preferred_element_type=jnp.float32,
                        precision=lax.Precision.HIGHEST)
                + cnts * b3r[...])
        pooled = sums / jnp.maximum(cnts, 1.0)
        mean = jnp.mean(pooled, axis=0, keepdims=True)
        var = jnp.mean((pooled - mean) ** 2, axis=0, keepdims=True)
        xn = (pooled - mean) * lax.rsqrt(var + 1e-5) * gr[...] + br[...]
        outr[...] = (jnp.dot(xn, Wlr[...],
                             preferred_element_type=jnp.float32,
                             precision=lax.Precision.HIGHEST)
                     + blr[...])


_NB = 10
_BN = N // _NB


def kernel(x, edge_index, edge_weight, batch, W1, b1, W2, b2, W3, b3,
           gamma, beta, Wlin, blin):
    pk3 = edge_index.reshape(2, NW, NCHUNK, K)
    w3 = edge_weight.reshape(NW, NCHUNK, K)

    u0p, tp = _make_edge_pass(True)(x, pk3, w3, batch)
    u0p = u0p.reshape(NC, N, D)
    tp = tp.reshape(NC, NS * TPW)[:, :G * N]

    h1, s0 = pl.pallas_call(
        _tc1_body,
        grid=(_NB,),
        in_specs=[
            pl.BlockSpec((NC, _BN, D), lambda i: (0, i, 0)),
            pl.BlockSpec((D, D), lambda i: (0, 0)),
            pl.BlockSpec((1, D), lambda i: (0, 0)),
        ],
        out_specs=[
            pl.BlockSpec((_BN, D), lambda i: (i, 0)),
            pl.BlockSpec((_BN, D), lambda i: (i, 0)),
        ],
        out_shape=[jax.ShapeDtypeStruct((N, D), jnp.float32)] * 2,
    )(u0p, W1, b1.reshape(1, D))

    u1p, = _make_edge_pass(False)(h1, pk3, w3)
    u1p = u1p.reshape(NC, N, D)

    t4 = tp.reshape(NC, G, _NB, _BN).transpose(0, 2, 1, 3)

    batch3 = batch.reshape(_NB, 1, _BN)
    C2 = 2 * D
    C3 = 3 * D
    out = pl.pallas_call(
        _tc2_body,
        grid=(_NB,),
        in_specs=[
            pl.BlockSpec((NC, _BN, D), lambda i: (0, i, 0)),
            pl.BlockSpec((_BN, D), lambda i: (i, 0)),
            pl.BlockSpec((_BN, D), lambda i: (i, 0)),
            pl.BlockSpec((NC, 1, G, _BN), lambda i: (0, i, 0, 0)),
            pl.BlockSpec((1, 1, _BN), lambda i: (i, 0, 0)),
            pl.BlockSpec((C2, C2), lambda i: (0, 0)),
            pl.BlockSpec((1, C2), lambda i: (0, 0)),
            pl.BlockSpec((C3, C3), lambda i: (0, 0)),
            pl.BlockSpec((1, C3), lambda i: (0, 0)),
            pl.BlockSpec((1, C3), lambda i: (0, 0)),
            pl.BlockSpec((1, C3), lambda i: (0, 0)),
            pl.BlockSpec((C3, 1), lambda i: (0, 0)),
            pl.BlockSpec((1, 1), lambda i: (0, 0)),
        ],
        out_specs=pl.BlockSpec((G, 1), lambda i: (0, 0)),
        out_shape=jax.ShapeDtypeStruct((G, 1), jnp.float32),
        scratch_shapes=[
            pltpu.VMEM((G, C2), jnp.float32),
            pltpu.VMEM((G, D), jnp.float32),
            pltpu.VMEM((G, D), jnp.float32),
        ],
    )(u1p, s0, h1, t4, batch3, W2, b2.reshape(1, C2),
      W3, b3.reshape(1, C3), gamma.reshape(1, C3), beta.reshape(1, C3),
      Wlin, blin.reshape(1, 1))
    return out
